# baseline (speedup 1.0000x reference)
import functools

import jax
import jax.numpy as jnp
from jax import lax
from jax.experimental import pallas as pl
from jax.experimental.pallas import tpu as pltpu
from jax.experimental.pallas import tpu_sc as plsc

_N = 10000
_E = 320000
_OD = 128
_RD = 16

_NC = 2
_NS = 16
_NW = _NC * _NS
_NSPLIT = 2
_EH = _E // _NSPLIT
_EW = _EH // _NW
_C = 40
_NCHUNK = _EW // _C
_ROWS_PER_TILE = 624
_TAIL_ROWS = _N - _NS * _ROWS_PER_TILE


def _pre_body(obj_ref, rw1a_ref, rw1b_ref, rb1_ref, ew1a_ref, eb1_ref,
              p_ref, q_ref, u_ref):
    obj = obj_ref[...]
    p_ref[...] = jnp.dot(obj, rw1a_ref[...], preferred_element_type=jnp.float32)
    q_ref[...] = (jnp.dot(obj, rw1b_ref[...], preferred_element_type=jnp.float32)
                  + rb1_ref[...])
    u_ref[...] = (jnp.dot(obj, ew1a_ref[...], preferred_element_type=jnp.float32)
                  + eb1_ref[...])


def _gather_body(p_hbm, q_hbm, snd_hbm, rcv_hbm, pg_hbm, qg_hbm,
                 sidx, ridx, prow, qrow, sem_p, sem_q):
    c = lax.axis_index("c")
    s = lax.axis_index("s")
    wid = s * _NC + c
    base = wid * _EW

    def body(j, carry):
        off = base + j * _C
        pltpu.sync_copy(snd_hbm.at[pl.ds(off, _C)], sidx)
        pltpu.sync_copy(rcv_hbm.at[pl.ds(off, _C)], ridx)
        cp_p = pltpu.async_copy(p_hbm.at[sidx], prow, sem_p)
        cp_q = pltpu.async_copy(q_hbm.at[ridx], qrow, sem_q)
        cp_p.wait()
        cp_q.wait()
        pltpu.sync_copy(prow, pg_hbm.at[pl.ds(off, _C)])
        pltpu.sync_copy(qrow, qg_hbm.at[pl.ds(off, _C)])
        return carry

    lax.fori_loop(0, _NCHUNK, body, 0)


def _edge_body(pg_ref, qg_ref, rel_ref, w1c_ref, w2_ref, b2_ref, out_ref):
    x = (pg_ref[...] + qg_ref[...]
         + jnp.dot(rel_ref[...], w1c_ref[...], preferred_element_type=jnp.float32))
    h = jnp.maximum(x, 0.0)
    out_ref[...] = (jnp.dot(h, w2_ref[...], preferred_element_type=jnp.float32)
                    + b2_ref[...])


def _scatter_body(eff_hbm, rcv_hbm, zeros_hbm, out_hbm, ridx, erow, acc, sem):
    c = lax.axis_index("c")
    s = lax.axis_index("s")
    wid = s * _NC + c

    pltpu.sync_copy(zeros_hbm.at[pl.ds(s * _ROWS_PER_TILE, _ROWS_PER_TILE)],
                    acc.at[pl.ds(s * _ROWS_PER_TILE, _ROWS_PER_TILE)])

    @pl.when(s == _NS - 1)
    def _zero_tail():
        pltpu.sync_copy(zeros_hbm.at[pl.ds(_NS * _ROWS_PER_TILE, _TAIL_ROWS)],
                        acc.at[pl.ds(_NS * _ROWS_PER_TILE, _TAIL_ROWS)])

    plsc.subcore_barrier()

    base = wid * _EW

    def body(j, carry):
        off = base + j * _C
        pltpu.sync_copy(rcv_hbm.at[pl.ds(off, _C)], ridx)
        pltpu.sync_copy(eff_hbm.at[pl.ds(off, _C)], erow)
        pltpu.sync_copy(erow, acc.at[ridx], add=True)
        return carry

    lax.fori_loop(0, _NCHUNK, body, 0)
    plsc.subcore_barrier()

    pltpu.sync_copy(acc.at[pl.ds(s * _ROWS_PER_TILE, _ROWS_PER_TILE)],
                    out_hbm.at[c, pl.ds(s * _ROWS_PER_TILE, _ROWS_PER_TILE)])

    @pl.when(s == _NS - 1)
    def _flush_tail():
        pltpu.sync_copy(acc.at[pl.ds(_NS * _ROWS_PER_TILE, _TAIL_ROWS)],
                        out_hbm.at[c, pl.ds(_NS * _ROWS_PER_TILE, _TAIL_ROWS)])


def _node_body(u_ref, a_ref, b_ref, ew1b_ref, ew2_ref, eb2_ref, out_ref):
    agg = a_ref[0] + a_ref[1] + b_ref[0] + b_ref[1]
    x = u_ref[...] + jnp.dot(agg, ew1b_ref[...], preferred_element_type=jnp.float32)
    h = jnp.maximum(x, 0.0)
    out_ref[...] = (jnp.dot(h, ew2_ref[...], preferred_element_type=jnp.float32)
                    + eb2_ref[...])


_NODE_BLK = 2000
_EDGE_BLK = 2000


def _full_spec(shape):
    return pl.BlockSpec(shape, lambda i: tuple(0 for _ in shape))


def kernel(objects, relations, senders, receivers,
           rW1, rb1, rW2, rb2,
           oW1, ob1, oW2, ob2,
           eW1, eb1, eW2, eb2):
    f32 = jnp.float32
    rW1a = rW1[:_OD]
    rW1b = rW1[_OD:2 * _OD]
    rW1c = rW1[2 * _OD:]
    eW1a = eW1[:_OD]
    eW1b = eW1[_OD:]
    rb1_2d = rb1.reshape(1, _OD)
    rb2_2d = rb2.reshape(1, _OD)
    eb1_2d = eb1.reshape(1, _OD)
    eb2_2d = eb2.reshape(1, _OD)

    n_grid = _N // _NODE_BLK
    row_spec = pl.BlockSpec((_NODE_BLK, _OD), lambda i: (i, 0))
    P, Q, U = pl.pallas_call(
        _pre_body,
        grid=(n_grid,),
        in_specs=[row_spec, _full_spec((_OD, _OD)), _full_spec((_OD, _OD)),
                  _full_spec((1, _OD)), _full_spec((_OD, _OD)),
                  _full_spec((1, _OD))],
        out_specs=[row_spec, row_spec, row_spec],
        out_shape=[jax.ShapeDtypeStruct((_N, _OD), f32)] * 3,
    )(objects, rW1a, rW1b, rb1_2d, eW1a, eb1_2d)

    mesh = plsc.VectorSubcoreMesh(core_axis_name="c", subcore_axis_name="s")
    gather = functools.partial(
        pl.kernel,
        mesh=mesh,
        out_type=[jax.ShapeDtypeStruct((_EH, _OD), f32),
                  jax.ShapeDtypeStruct((_EH, _OD), f32)],
        scratch_types=[
            pltpu.VMEM((_C,), jnp.int32),
            pltpu.VMEM((_C,), jnp.int32),
            pltpu.VMEM((_C, _OD), f32),
            pltpu.VMEM((_C, _OD), f32),
            pltpu.SemaphoreType.DMA,
            pltpu.SemaphoreType.DMA,
        ],
    )(_gather_body)

    scatter = functools.partial(
        pl.kernel,
        mesh=mesh,
        out_type=jax.ShapeDtypeStruct((_NC, _N, _OD), f32),
        scratch_types=[
            pltpu.VMEM((_C,), jnp.int32),
            pltpu.VMEM((_C, _OD), f32),
            pltpu.VMEM_SHARED((_N, _OD), f32),
            pltpu.SemaphoreType.DMA,
        ],
    )(_scatter_body)

    e_grid = _EH // _EDGE_BLK
    erow_spec = pl.BlockSpec((_EDGE_BLK, _OD), lambda i: (i, 0))
    rel_spec = pl.BlockSpec((_EDGE_BLK, _RD), lambda i: (i, 0))
    edge_mlp = pl.pallas_call(
        _edge_body,
        grid=(e_grid,),
        in_specs=[erow_spec, erow_spec, rel_spec, _full_spec((_RD, _OD)),
                  _full_spec((_OD, _OD)), _full_spec((1, _OD))],
        out_specs=erow_spec,
        out_shape=jax.ShapeDtypeStruct((_EH, _OD), f32),
    )

    zeros = jnp.zeros((_N, _OD), f32)
    aggs = []
    for h in range(_NSPLIT):
        lo, hi = h * _EH, (h + 1) * _EH
        snd_h = senders[lo:hi]
        rcv_h = receivers[lo:hi]
        Pg, Qg = gather(P, Q, snd_h, rcv_h)
        eff = edge_mlp(Pg, Qg, relations[lo:hi], rW1c, rW2, rb2_2d)
        aggs.append(scatter(eff, rcv_h, zeros))

    agg_spec = pl.BlockSpec((_NC, _NODE_BLK, _OD), lambda i: (0, i, 0))
    out = pl.pallas_call(
        _node_body,
        grid=(n_grid,),
        in_specs=[row_spec, agg_spec, agg_spec, _full_spec((_OD, _OD)),
                  _full_spec((_OD, _OD)), _full_spec((1, _OD))],
        out_specs=row_spec,
        out_shape=jax.ShapeDtypeStruct((_N, _OD), f32),
    )(U, aggs[0], aggs[1], eW1b, eW2, eb2_2d)
    return out


# speedup vs baseline: 1.5506x; 1.5506x over previous
import functools

import jax
import jax.numpy as jnp
from jax import lax
from jax.experimental import pallas as pl
from jax.experimental.pallas import tpu as pltpu
from jax.experimental.pallas import tpu_sc as plsc

_N = 10000
_E = 320000
_OD = 128
_RD = 16

_NC = 2
_NS = 16
_NW = _NC * _NS
_NSPLIT = 2
_EH = _E // _NSPLIT
_EW = _EH // _NW
_C = 80
_NCHUNK = _EW // _C
_CT = _EW - _NCHUNK * _C
_ROWS_PER_TILE = 624
_TAIL_ROWS = _N - _NS * _ROWS_PER_TILE


def _pre_body(obj_ref, rw1a_ref, rw1b_ref, rb1_ref, ew1a_ref, eb1_ref,
              p_ref, q_ref, u_ref):
    obj = obj_ref[...]
    p_ref[...] = jnp.dot(obj, rw1a_ref[...], preferred_element_type=jnp.float32)
    q_ref[...] = (jnp.dot(obj, rw1b_ref[...], preferred_element_type=jnp.float32)
                  + rb1_ref[...])
    u_ref[...] = (jnp.dot(obj, ew1a_ref[...], preferred_element_type=jnp.float32)
                  + eb1_ref[...])


def _gather_body(p_hbm, q_hbm, snd_hbm, rcv_hbm, pg_hbm, qg_hbm,
                 sall, rall, prow, qrow, sem_p, sem_q):
    c = lax.axis_index("c")
    s = lax.axis_index("s")
    wid = s * _NC + c
    base = wid * _EW

    pltpu.sync_copy(snd_hbm.at[pl.ds(base, _EW)], sall)
    pltpu.sync_copy(rcv_hbm.at[pl.ds(base, _EW)], rall)

    def body(j, carry):
        off = base + j * _C
        loc = j * _C
        cp_p = pltpu.async_copy(p_hbm.at[sall.at[pl.ds(loc, _C)]], prow, sem_p)
        cp_q = pltpu.async_copy(q_hbm.at[rall.at[pl.ds(loc, _C)]], qrow, sem_q)
        cp_p.wait()
        cp_q.wait()
        pltpu.sync_copy(prow, pg_hbm.at[pl.ds(off, _C)])
        pltpu.sync_copy(qrow, qg_hbm.at[pl.ds(off, _C)])
        return carry

    lax.fori_loop(0, _NCHUNK, body, 0)

    toff = base + _NCHUNK * _C
    tloc = _NCHUNK * _C
    cp_p = pltpu.async_copy(p_hbm.at[sall.at[pl.ds(tloc, _CT)]],
                            prow.at[pl.ds(0, _CT)], sem_p)
    cp_q = pltpu.async_copy(q_hbm.at[rall.at[pl.ds(tloc, _CT)]],
                            qrow.at[pl.ds(0, _CT)], sem_q)
    cp_p.wait()
    cp_q.wait()
    pltpu.sync_copy(prow.at[pl.ds(0, _CT)], pg_hbm.at[pl.ds(toff, _CT)])
    pltpu.sync_copy(qrow.at[pl.ds(0, _CT)], qg_hbm.at[pl.ds(toff, _CT)])


def _edge_body(pg_ref, qg_ref, rel_ref, w1c_ref, w2_ref, b2_ref, out_ref):
    x = (pg_ref[...] + qg_ref[...]
         + jnp.dot(rel_ref[...], w1c_ref[...], preferred_element_type=jnp.float32))
    h = jnp.maximum(x, 0.0)
    out_ref[...] = (jnp.dot(h, w2_ref[...], preferred_element_type=jnp.float32)
                    + b2_ref[...])


def _scatter_body(eff_hbm, rcv_hbm, zeros_hbm, out_hbm, ridx, ridx_t, erow, acc,
                  sem):
    c = lax.axis_index("c")
    s = lax.axis_index("s")
    wid = s * _NC + c

    pltpu.sync_copy(zeros_hbm.at[pl.ds(s * _ROWS_PER_TILE, _ROWS_PER_TILE)],
                    acc.at[pl.ds(s * _ROWS_PER_TILE, _ROWS_PER_TILE)])

    @pl.when(s == _NS - 1)
    def _zero_tail():
        pltpu.sync_copy(zeros_hbm.at[pl.ds(_NS * _ROWS_PER_TILE, _TAIL_ROWS)],
                        acc.at[pl.ds(_NS * _ROWS_PER_TILE, _TAIL_ROWS)])

    plsc.subcore_barrier()

    base = wid * _EW

    def body(j, carry):
        off = base + j * _C
        pltpu.sync_copy(rcv_hbm.at[pl.ds(off, _C)], ridx)
        pltpu.sync_copy(eff_hbm.at[pl.ds(off, _C)], erow)
        pltpu.sync_copy(erow, acc.at[ridx], add=True)
        return carry

    lax.fori_loop(0, _NCHUNK, body, 0)

    toff = base + _NCHUNK * _C
    pltpu.sync_copy(rcv_hbm.at[pl.ds(toff, _CT)], ridx_t)
    pltpu.sync_copy(eff_hbm.at[pl.ds(toff, _CT)], erow.at[pl.ds(0, _CT)])
    pltpu.sync_copy(erow.at[pl.ds(0, _CT)], acc.at[ridx_t], add=True)
    plsc.subcore_barrier()

    pltpu.sync_copy(acc.at[pl.ds(s * _ROWS_PER_TILE, _ROWS_PER_TILE)],
                    out_hbm.at[c, pl.ds(s * _ROWS_PER_TILE, _ROWS_PER_TILE)])

    @pl.when(s == _NS - 1)
    def _flush_tail():
        pltpu.sync_copy(acc.at[pl.ds(_NS * _ROWS_PER_TILE, _TAIL_ROWS)],
                        out_hbm.at[c, pl.ds(_NS * _ROWS_PER_TILE, _TAIL_ROWS)])


def _node_body(u_ref, a_ref, b_ref, ew1b_ref, ew2_ref, eb2_ref, out_ref):
    agg = a_ref[0] + a_ref[1] + b_ref[0] + b_ref[1]
    x = u_ref[...] + jnp.dot(agg, ew1b_ref[...], preferred_element_type=jnp.float32)
    h = jnp.maximum(x, 0.0)
    out_ref[...] = (jnp.dot(h, ew2_ref[...], preferred_element_type=jnp.float32)
                    + eb2_ref[...])


_NODE_BLK = 2000
_EDGE_BLK = 2000


def _full_spec(shape):
    return pl.BlockSpec(shape, lambda i: tuple(0 for _ in shape))


def kernel(objects, relations, senders, receivers,
           rW1, rb1, rW2, rb2,
           oW1, ob1, oW2, ob2,
           eW1, eb1, eW2, eb2):
    f32 = jnp.float32
    rW1a = rW1[:_OD]
    rW1b = rW1[_OD:2 * _OD]
    rW1c = rW1[2 * _OD:]
    eW1a = eW1[:_OD]
    eW1b = eW1[_OD:]
    rb1_2d = rb1.reshape(1, _OD)
    rb2_2d = rb2.reshape(1, _OD)
    eb1_2d = eb1.reshape(1, _OD)
    eb2_2d = eb2.reshape(1, _OD)

    n_grid = _N // _NODE_BLK
    row_spec = pl.BlockSpec((_NODE_BLK, _OD), lambda i: (i, 0))
    P, Q, U = pl.pallas_call(
        _pre_body,
        grid=(n_grid,),
        in_specs=[row_spec, _full_spec((_OD, _OD)), _full_spec((_OD, _OD)),
                  _full_spec((1, _OD)), _full_spec((_OD, _OD)),
                  _full_spec((1, _OD))],
        out_specs=[row_spec, row_spec, row_spec],
        out_shape=[jax.ShapeDtypeStruct((_N, _OD), f32)] * 3,
    )(objects, rW1a, rW1b, rb1_2d, eW1a, eb1_2d)

    mesh = plsc.VectorSubcoreMesh(core_axis_name="c", subcore_axis_name="s")
    gather = functools.partial(
        pl.kernel,
        mesh=mesh,
        out_type=[jax.ShapeDtypeStruct((_EH, _OD), f32),
                  jax.ShapeDtypeStruct((_EH, _OD), f32)],
        scratch_types=[
            pltpu.VMEM((_EW,), jnp.int32),
            pltpu.VMEM((_EW,), jnp.int32),
            pltpu.VMEM((_C, _OD), f32),
            pltpu.VMEM((_C, _OD), f32),
            pltpu.SemaphoreType.DMA,
            pltpu.SemaphoreType.DMA,
        ],
    )(_gather_body)

    scatter = functools.partial(
        pl.kernel,
        mesh=mesh,
        out_type=jax.ShapeDtypeStruct((_NC, _N, _OD), f32),
        scratch_types=[
            pltpu.VMEM((_C,), jnp.int32),
            pltpu.VMEM((_CT,), jnp.int32),
            pltpu.VMEM((_C, _OD), f32),
            pltpu.VMEM_SHARED((_N, _OD), f32),
            pltpu.SemaphoreType.DMA,
        ],
    )(_scatter_body)

    e_grid = _EH // _EDGE_BLK
    erow_spec = pl.BlockSpec((_EDGE_BLK, _OD), lambda i: (i, 0))
    rel_spec = pl.BlockSpec((_EDGE_BLK, _RD), lambda i: (i, 0))
    edge_mlp = pl.pallas_call(
        _edge_body,
        grid=(e_grid,),
        in_specs=[erow_spec, erow_spec, rel_spec, _full_spec((_RD, _OD)),
                  _full_spec((_OD, _OD)), _full_spec((1, _OD))],
        out_specs=erow_spec,
        out_shape=jax.ShapeDtypeStruct((_EH, _OD), f32),
    )

    zeros = jnp.zeros((_N, _OD), f32)
    aggs = []
    for h in range(_NSPLIT):
        lo, hi = h * _EH, (h + 1) * _EH
        snd_h = senders[lo:hi]
        rcv_h = receivers[lo:hi]
        Pg, Qg = gather(P, Q, snd_h, rcv_h)
        eff = edge_mlp(Pg, Qg, relations[lo:hi], rW1c, rW2, rb2_2d)
        aggs.append(scatter(eff, rcv_h, zeros))

    agg_spec = pl.BlockSpec((_NC, _NODE_BLK, _OD), lambda i: (0, i, 0))
    out = pl.pallas_call(
        _node_body,
        grid=(n_grid,),
        in_specs=[row_spec, agg_spec, agg_spec, _full_spec((_OD, _OD)),
                  _full_spec((_OD, _OD)), _full_spec((1, _OD))],
        out_specs=row_spec,
        out_shape=jax.ShapeDtypeStruct((_N, _OD), f32),
    )(U, aggs[0], aggs[1], eW1b, eW2, eb2_2d)
    return out


# speedup vs baseline: 1.7642x; 1.1378x over previous
import functools

import jax
import jax.numpy as jnp
from jax import lax
from jax.experimental import pallas as pl
from jax.experimental.pallas import tpu as pltpu
from jax.experimental.pallas import tpu_sc as plsc

_N = 10000
_E = 320000
_OD = 128
_RD = 16

_NC = 2
_NS = 16
_NW = _NC * _NS
_NSPLIT = 2
_EH = _E // _NSPLIT
_EW = _EH // _NW
_C = 80
_NCHUNK = _EW // _C
_CT = _EW - _NCHUNK * _C
_ROWS_PER_TILE = 624
_TAIL_ROWS = _N - _NS * _ROWS_PER_TILE


def _pre_body(obj_ref, rw1a_ref, rw1b_ref, rb1_ref, ew1a_ref, eb1_ref,
              p_ref, q_ref, u_ref):
    obj = obj_ref[...]
    p_ref[...] = jnp.dot(obj, rw1a_ref[...], preferred_element_type=jnp.float32)
    q_ref[...] = (jnp.dot(obj, rw1b_ref[...], preferred_element_type=jnp.float32)
                  + rb1_ref[...])
    u_ref[...] = (jnp.dot(obj, ew1a_ref[...], preferred_element_type=jnp.float32)
                  + eb1_ref[...])


def _add_rows(pbuf, qbuf, sbuf, rows):
    def row(r, carry):
        for g in range(_OD // 16):
            sl = pl.ds(g * 16, 16)
            sbuf[r, sl] = pbuf[r, sl] + qbuf[r, sl]
        return carry

    lax.fori_loop(0, rows, row, 0)


def _gather_body(p_hbm, q_hbm, snd_hbm, rcv_hbm, s_hbm,
                 sall, rall, pbuf0, qbuf0, sbuf0, pbuf1, qbuf1, sbuf1,
                 semg0, semg1, semo0, semo1):
    c = lax.axis_index("c")
    s = lax.axis_index("s")
    wid = s * _NC + c
    base = wid * _EW

    pltpu.sync_copy(snd_hbm.at[pl.ds(base, _EW)], sall)
    pltpu.sync_copy(rcv_hbm.at[pl.ds(base, _EW)], rall)

    npair = _NCHUNK // 2

    def pair(i, carry):
        a = 2 * i * _C
        b = a + _C
        cpa_p = pltpu.async_copy(p_hbm.at[sall.at[pl.ds(a, _C)]], pbuf0, semg0)
        cpa_q = pltpu.async_copy(q_hbm.at[rall.at[pl.ds(a, _C)]], qbuf0, semg0)
        cpb_p = pltpu.async_copy(p_hbm.at[sall.at[pl.ds(b, _C)]], pbuf1, semg1)
        cpb_q = pltpu.async_copy(q_hbm.at[rall.at[pl.ds(b, _C)]], qbuf1, semg1)
        cpa_p.wait()
        cpa_q.wait()
        _add_rows(pbuf0, qbuf0, sbuf0, _C)
        wba = pltpu.async_copy(sbuf0, s_hbm.at[pl.ds(base + a, _C)], semo0)
        cpb_p.wait()
        cpb_q.wait()
        _add_rows(pbuf1, qbuf1, sbuf1, _C)
        wbb = pltpu.async_copy(sbuf1, s_hbm.at[pl.ds(base + b, _C)], semo1)
        wba.wait()
        wbb.wait()
        return carry

    lax.fori_loop(0, npair, pair, 0)

    toff = _NCHUNK * _C
    cp_p = pltpu.async_copy(p_hbm.at[sall.at[pl.ds(toff, _CT)]],
                            pbuf0.at[pl.ds(0, _CT)], semg0)
    cp_q = pltpu.async_copy(q_hbm.at[rall.at[pl.ds(toff, _CT)]],
                            qbuf0.at[pl.ds(0, _CT)], semg0)
    cp_p.wait()
    cp_q.wait()
    _add_rows(pbuf0, qbuf0, sbuf0, _CT)
    pltpu.sync_copy(sbuf0.at[pl.ds(0, _CT)], s_hbm.at[pl.ds(base + toff, _CT)])


def _edge_body(s_ref, rel_ref, w1c_ref, w2_ref, b2_ref, out_ref):
    x = (s_ref[...]
         + jnp.dot(rel_ref[...], w1c_ref[...], preferred_element_type=jnp.float32))
    h = jnp.maximum(x, 0.0)
    out_ref[...] = (jnp.dot(h, w2_ref[...], preferred_element_type=jnp.float32)
                    + b2_ref[...])


def _scatter_body(eff_hbm, rcv_hbm, zeros_hbm, out_hbm, ridx, ridx_t, erow, acc,
                  sem):
    c = lax.axis_index("c")
    s = lax.axis_index("s")
    wid = s * _NC + c

    pltpu.sync_copy(zeros_hbm.at[pl.ds(s * _ROWS_PER_TILE, _ROWS_PER_TILE)],
                    acc.at[pl.ds(s * _ROWS_PER_TILE, _ROWS_PER_TILE)])

    @pl.when(s == _NS - 1)
    def _zero_tail():
        pltpu.sync_copy(zeros_hbm.at[pl.ds(_NS * _ROWS_PER_TILE, _TAIL_ROWS)],
                        acc.at[pl.ds(_NS * _ROWS_PER_TILE, _TAIL_ROWS)])

    plsc.subcore_barrier()

    base = wid * _EW

    def body(j, carry):
        off = base + j * _C
        pltpu.sync_copy(rcv_hbm.at[pl.ds(off, _C)], ridx)
        pltpu.sync_copy(eff_hbm.at[pl.ds(off, _C)], erow)
        pltpu.sync_copy(erow, acc.at[ridx], add=True)
        return carry

    lax.fori_loop(0, _NCHUNK, body, 0)

    toff = base + _NCHUNK * _C
    pltpu.sync_copy(rcv_hbm.at[pl.ds(toff, _CT)], ridx_t)
    pltpu.sync_copy(eff_hbm.at[pl.ds(toff, _CT)], erow.at[pl.ds(0, _CT)])
    pltpu.sync_copy(erow.at[pl.ds(0, _CT)], acc.at[ridx_t], add=True)
    plsc.subcore_barrier()

    pltpu.sync_copy(acc.at[pl.ds(s * _ROWS_PER_TILE, _ROWS_PER_TILE)],
                    out_hbm.at[c, pl.ds(s * _ROWS_PER_TILE, _ROWS_PER_TILE)])

    @pl.when(s == _NS - 1)
    def _flush_tail():
        pltpu.sync_copy(acc.at[pl.ds(_NS * _ROWS_PER_TILE, _TAIL_ROWS)],
                        out_hbm.at[c, pl.ds(_NS * _ROWS_PER_TILE, _TAIL_ROWS)])


def _node_body(u_ref, a_ref, b_ref, ew1b_ref, ew2_ref, eb2_ref, out_ref):
    agg = a_ref[0] + a_ref[1] + b_ref[0] + b_ref[1]
    x = u_ref[...] + jnp.dot(agg, ew1b_ref[...], preferred_element_type=jnp.float32)
    h = jnp.maximum(x, 0.0)
    out_ref[...] = (jnp.dot(h, ew2_ref[...], preferred_element_type=jnp.float32)
                    + eb2_ref[...])


_NODE_BLK = 2000
_EDGE_BLK = 2000


def _full_spec(shape):
    return pl.BlockSpec(shape, lambda i: tuple(0 for _ in shape))


def kernel(objects, relations, senders, receivers,
           rW1, rb1, rW2, rb2,
           oW1, ob1, oW2, ob2,
           eW1, eb1, eW2, eb2):
    f32 = jnp.float32
    rW1a = rW1[:_OD]
    rW1b = rW1[_OD:2 * _OD]
    rW1c = rW1[2 * _OD:]
    eW1a = eW1[:_OD]
    eW1b = eW1[_OD:]
    rb1_2d = rb1.reshape(1, _OD)
    rb2_2d = rb2.reshape(1, _OD)
    eb1_2d = eb1.reshape(1, _OD)
    eb2_2d = eb2.reshape(1, _OD)

    n_grid = _N // _NODE_BLK
    row_spec = pl.BlockSpec((_NODE_BLK, _OD), lambda i: (i, 0))
    P, Q, U = pl.pallas_call(
        _pre_body,
        grid=(n_grid,),
        in_specs=[row_spec, _full_spec((_OD, _OD)), _full_spec((_OD, _OD)),
                  _full_spec((1, _OD)), _full_spec((_OD, _OD)),
                  _full_spec((1, _OD))],
        out_specs=[row_spec, row_spec, row_spec],
        out_shape=[jax.ShapeDtypeStruct((_N, _OD), f32)] * 3,
    )(objects, rW1a, rW1b, rb1_2d, eW1a, eb1_2d)

    mesh = plsc.VectorSubcoreMesh(core_axis_name="c", subcore_axis_name="s")
    gather = functools.partial(
        pl.kernel,
        mesh=mesh,
        out_type=jax.ShapeDtypeStruct((_EH, _OD), f32),
        scratch_types=[
            pltpu.VMEM((_EW,), jnp.int32),
            pltpu.VMEM((_EW,), jnp.int32),
            pltpu.VMEM((_C, _OD), f32),
            pltpu.VMEM((_C, _OD), f32),
            pltpu.VMEM((_C, _OD), f32),
            pltpu.VMEM((_C, _OD), f32),
            pltpu.VMEM((_C, _OD), f32),
            pltpu.VMEM((_C, _OD), f32),
            pltpu.SemaphoreType.DMA,
            pltpu.SemaphoreType.DMA,
            pltpu.SemaphoreType.DMA,
            pltpu.SemaphoreType.DMA,
        ],
    )(_gather_body)

    scatter = functools.partial(
        pl.kernel,
        mesh=mesh,
        out_type=jax.ShapeDtypeStruct((_NC, _N, _OD), f32),
        scratch_types=[
            pltpu.VMEM((_C,), jnp.int32),
            pltpu.VMEM((_CT,), jnp.int32),
            pltpu.VMEM((_C, _OD), f32),
            pltpu.VMEM_SHARED((_N, _OD), f32),
            pltpu.SemaphoreType.DMA,
        ],
    )(_scatter_body)

    e_grid = _EH // _EDGE_BLK
    erow_spec = pl.BlockSpec((_EDGE_BLK, _OD), lambda i: (i, 0))
    rel_spec = pl.BlockSpec((_EDGE_BLK, _RD), lambda i: (i, 0))
    edge_mlp = pl.pallas_call(
        _edge_body,
        grid=(e_grid,),
        in_specs=[erow_spec, rel_spec, _full_spec((_RD, _OD)),
                  _full_spec((_OD, _OD)), _full_spec((1, _OD))],
        out_specs=erow_spec,
        out_shape=jax.ShapeDtypeStruct((_EH, _OD), f32),
    )

    zeros = jnp.zeros((_N, _OD), f32)
    aggs = []
    for h in range(_NSPLIT):
        lo, hi = h * _EH, (h + 1) * _EH
        snd_h = senders[lo:hi]
        rcv_h = receivers[lo:hi]
        S = gather(P, Q, snd_h, rcv_h)
        eff = edge_mlp(S, relations[lo:hi], rW1c, rW2, rb2_2d)
        aggs.append(scatter(eff, rcv_h, zeros))

    agg_spec = pl.BlockSpec((_NC, _NODE_BLK, _OD), lambda i: (0, i, 0))
    out = pl.pallas_call(
        _node_body,
        grid=(n_grid,),
        in_specs=[row_spec, agg_spec, agg_spec, _full_spec((_OD, _OD)),
                  _full_spec((_OD, _OD)), _full_spec((1, _OD))],
        out_specs=row_spec,
        out_shape=jax.ShapeDtypeStruct((_N, _OD), f32),
    )(U, aggs[0], aggs[1], eW1b, eW2, eb2_2d)
    return out


# speedup vs baseline: 1.9247x; 1.0910x over previous
import functools

import jax
import jax.numpy as jnp
from jax import lax
from jax.experimental import pallas as pl
from jax.experimental.pallas import tpu as pltpu
from jax.experimental.pallas import tpu_sc as plsc

_N = 10000
_E = 320000
_OD = 128
_RD = 16

_NC = 2
_NS = 16
_NW = _NC * _NS
_NSPLIT = 2
_EH = _E // _NSPLIT
_EW = _EH // _NW
_C = 80
_NCHUNK = _EW // _C
_CT = _EW - _NCHUNK * _C
_ROWS_PER_TILE = 624
_TAIL_ROWS = _N - _NS * _ROWS_PER_TILE


def _pre_body(obj_ref, rw1a_ref, rw1b_ref, rb1_ref, ew1a_ref, eb1_ref,
              p_ref, q_ref, u_ref):
    obj = obj_ref[...]
    p_ref[...] = jnp.dot(obj, rw1a_ref[...], preferred_element_type=jnp.float32)
    q_ref[...] = (jnp.dot(obj, rw1b_ref[...], preferred_element_type=jnp.float32)
                  + rb1_ref[...])
    u_ref[...] = (jnp.dot(obj, ew1a_ref[...], preferred_element_type=jnp.float32)
                  + eb1_ref[...])


def _add_rows(pbuf, qbuf, sbuf, rows):
    def row(r, carry):
        for g in range(_OD // 16):
            sl = pl.ds(g * 16, 16)
            sbuf[r, sl] = pbuf[r, sl] + qbuf[r, sl]
        return carry

    lax.fori_loop(0, rows, row, 0)


def _gather_body(p_hbm, q_hbm, snd_hbm, rcv_hbm, s_hbm,
                 sall, rall, pbuf0, qbuf0, sbuf0, pbuf1, qbuf1, sbuf1,
                 semg0, semg1, semo0, semo1):
    c = lax.axis_index("c")
    s = lax.axis_index("s")
    wid = s * _NC + c
    base = wid * _EW

    pltpu.sync_copy(snd_hbm.at[pl.ds(base, _EW)], sall)
    pltpu.sync_copy(rcv_hbm.at[pl.ds(base, _EW)], rall)

    npair = _NCHUNK // 2

    def pair(i, carry):
        a = 2 * i * _C
        b = a + _C
        cpa_p = pltpu.async_copy(p_hbm.at[sall.at[pl.ds(a, _C)]], pbuf0, semg0)
        cpa_q = pltpu.async_copy(q_hbm.at[rall.at[pl.ds(a, _C)]], qbuf0, semg0)
        cpb_p = pltpu.async_copy(p_hbm.at[sall.at[pl.ds(b, _C)]], pbuf1, semg1)
        cpb_q = pltpu.async_copy(q_hbm.at[rall.at[pl.ds(b, _C)]], qbuf1, semg1)
        cpa_p.wait()
        cpa_q.wait()
        _add_rows(pbuf0, qbuf0, sbuf0, _C)
        wba = pltpu.async_copy(sbuf0, s_hbm.at[pl.ds(base + a, _C)], semo0)
        cpb_p.wait()
        cpb_q.wait()
        _add_rows(pbuf1, qbuf1, sbuf1, _C)
        wbb = pltpu.async_copy(sbuf1, s_hbm.at[pl.ds(base + b, _C)], semo1)
        wba.wait()
        wbb.wait()
        return carry

    lax.fori_loop(0, npair, pair, 0)

    toff = _NCHUNK * _C
    cp_p = pltpu.async_copy(p_hbm.at[sall.at[pl.ds(toff, _CT)]],
                            pbuf0.at[pl.ds(0, _CT)], semg0)
    cp_q = pltpu.async_copy(q_hbm.at[rall.at[pl.ds(toff, _CT)]],
                            qbuf0.at[pl.ds(0, _CT)], semg0)
    cp_p.wait()
    cp_q.wait()
    _add_rows(pbuf0, qbuf0, sbuf0, _CT)
    pltpu.sync_copy(sbuf0.at[pl.ds(0, _CT)], s_hbm.at[pl.ds(base + toff, _CT)])


def _edge_body(s_ref, rel_ref, w1c_ref, w2_ref, b2_ref, out_ref):
    x = (s_ref[...]
         + jnp.dot(rel_ref[...], w1c_ref[...], preferred_element_type=jnp.float32))
    h = jnp.maximum(x, 0.0)
    out_ref[...] = (jnp.dot(h, w2_ref[...], preferred_element_type=jnp.float32)
                    + b2_ref[...])


def _scatter_body(eff_hbm, rcv_hbm, zeros_hbm, out_hbm, ridx0, ridx1, ridx_t,
                  ebuf0, ebuf1, acc, seml0, seml1, sema0, sema1):
    c = lax.axis_index("c")
    s = lax.axis_index("s")
    wid = s * _NC + c

    pltpu.sync_copy(zeros_hbm.at[pl.ds(s * _ROWS_PER_TILE, _ROWS_PER_TILE)],
                    acc.at[pl.ds(s * _ROWS_PER_TILE, _ROWS_PER_TILE)])

    @pl.when(s == _NS - 1)
    def _zero_tail():
        pltpu.sync_copy(zeros_hbm.at[pl.ds(_NS * _ROWS_PER_TILE, _TAIL_ROWS)],
                        acc.at[pl.ds(_NS * _ROWS_PER_TILE, _TAIL_ROWS)])

    plsc.subcore_barrier()

    base = wid * _EW
    npair = _NCHUNK // 2

    def pair(i, carry):
        a = base + 2 * i * _C
        b = a + _C
        pltpu.sync_copy(rcv_hbm.at[pl.ds(a, _C)], ridx0)
        cpa = pltpu.async_copy(eff_hbm.at[pl.ds(a, _C)], ebuf0, seml0)
        pltpu.sync_copy(rcv_hbm.at[pl.ds(b, _C)], ridx1)
        cpb = pltpu.async_copy(eff_hbm.at[pl.ds(b, _C)], ebuf1, seml1)
        cpa.wait()
        adda = pltpu.async_copy(ebuf0, acc.at[ridx0], sema0, add=True)
        cpb.wait()
        addb = pltpu.async_copy(ebuf1, acc.at[ridx1], sema1, add=True)
        adda.wait()
        addb.wait()
        return carry

    lax.fori_loop(0, npair, pair, 0)

    toff = base + _NCHUNK * _C
    pltpu.sync_copy(rcv_hbm.at[pl.ds(toff, _CT)], ridx_t)
    pltpu.sync_copy(eff_hbm.at[pl.ds(toff, _CT)], ebuf0.at[pl.ds(0, _CT)])
    pltpu.sync_copy(ebuf0.at[pl.ds(0, _CT)], acc.at[ridx_t], add=True)
    plsc.subcore_barrier()

    pltpu.sync_copy(acc.at[pl.ds(s * _ROWS_PER_TILE, _ROWS_PER_TILE)],
                    out_hbm.at[c, pl.ds(s * _ROWS_PER_TILE, _ROWS_PER_TILE)])

    @pl.when(s == _NS - 1)
    def _flush_tail():
        pltpu.sync_copy(acc.at[pl.ds(_NS * _ROWS_PER_TILE, _TAIL_ROWS)],
                        out_hbm.at[c, pl.ds(_NS * _ROWS_PER_TILE, _TAIL_ROWS)])


def _node_body(u_ref, a_ref, b_ref, ew1b_ref, ew2_ref, eb2_ref, out_ref):
    agg = a_ref[0] + a_ref[1] + b_ref[0] + b_ref[1]
    x = u_ref[...] + jnp.dot(agg, ew1b_ref[...], preferred_element_type=jnp.float32)
    h = jnp.maximum(x, 0.0)
    out_ref[...] = (jnp.dot(h, ew2_ref[...], preferred_element_type=jnp.float32)
                    + eb2_ref[...])


_NODE_BLK = 2000
_EDGE_BLK = 2000


def _full_spec(shape):
    return pl.BlockSpec(shape, lambda i: tuple(0 for _ in shape))


def kernel(objects, relations, senders, receivers,
           rW1, rb1, rW2, rb2,
           oW1, ob1, oW2, ob2,
           eW1, eb1, eW2, eb2):
    f32 = jnp.float32
    rW1a = rW1[:_OD]
    rW1b = rW1[_OD:2 * _OD]
    rW1c = rW1[2 * _OD:]
    eW1a = eW1[:_OD]
    eW1b = eW1[_OD:]
    rb1_2d = rb1.reshape(1, _OD)
    rb2_2d = rb2.reshape(1, _OD)
    eb1_2d = eb1.reshape(1, _OD)
    eb2_2d = eb2.reshape(1, _OD)

    n_grid = _N // _NODE_BLK
    row_spec = pl.BlockSpec((_NODE_BLK, _OD), lambda i: (i, 0))
    P, Q, U = pl.pallas_call(
        _pre_body,
        grid=(n_grid,),
        in_specs=[row_spec, _full_spec((_OD, _OD)), _full_spec((_OD, _OD)),
                  _full_spec((1, _OD)), _full_spec((_OD, _OD)),
                  _full_spec((1, _OD))],
        out_specs=[row_spec, row_spec, row_spec],
        out_shape=[jax.ShapeDtypeStruct((_N, _OD), f32)] * 3,
    )(objects, rW1a, rW1b, rb1_2d, eW1a, eb1_2d)

    mesh = plsc.VectorSubcoreMesh(core_axis_name="c", subcore_axis_name="s")
    gather = functools.partial(
        pl.kernel,
        mesh=mesh,
        out_type=jax.ShapeDtypeStruct((_EH, _OD), f32),
        scratch_types=[
            pltpu.VMEM((_EW,), jnp.int32),
            pltpu.VMEM((_EW,), jnp.int32),
            pltpu.VMEM((_C, _OD), f32),
            pltpu.VMEM((_C, _OD), f32),
            pltpu.VMEM((_C, _OD), f32),
            pltpu.VMEM((_C, _OD), f32),
            pltpu.VMEM((_C, _OD), f32),
            pltpu.VMEM((_C, _OD), f32),
            pltpu.SemaphoreType.DMA,
            pltpu.SemaphoreType.DMA,
            pltpu.SemaphoreType.DMA,
            pltpu.SemaphoreType.DMA,
        ],
    )(_gather_body)

    scatter = functools.partial(
        pl.kernel,
        mesh=mesh,
        out_type=jax.ShapeDtypeStruct((_NC, _N, _OD), f32),
        scratch_types=[
            pltpu.VMEM((_C,), jnp.int32),
            pltpu.VMEM((_C,), jnp.int32),
            pltpu.VMEM((_CT,), jnp.int32),
            pltpu.VMEM((_C, _OD), f32),
            pltpu.VMEM((_C, _OD), f32),
            pltpu.VMEM_SHARED((_N, _OD), f32),
            pltpu.SemaphoreType.DMA,
            pltpu.SemaphoreType.DMA,
            pltpu.SemaphoreType.DMA,
            pltpu.SemaphoreType.DMA,
        ],
    )(_scatter_body)

    e_grid = _EH // _EDGE_BLK
    erow_spec = pl.BlockSpec((_EDGE_BLK, _OD), lambda i: (i, 0))
    rel_spec = pl.BlockSpec((_EDGE_BLK, _RD), lambda i: (i, 0))
    edge_mlp = pl.pallas_call(
        _edge_body,
        grid=(e_grid,),
        in_specs=[erow_spec, rel_spec, _full_spec((_RD, _OD)),
                  _full_spec((_OD, _OD)), _full_spec((1, _OD))],
        out_specs=erow_spec,
        out_shape=jax.ShapeDtypeStruct((_EH, _OD), f32),
    )

    zeros = jnp.zeros((_N, _OD), f32)
    aggs = []
    for h in range(_NSPLIT):
        lo, hi = h * _EH, (h + 1) * _EH
        snd_h = senders[lo:hi]
        rcv_h = receivers[lo:hi]
        S = gather(P, Q, snd_h, rcv_h)
        eff = edge_mlp(S, relations[lo:hi], rW1c, rW2, rb2_2d)
        aggs.append(scatter(eff, rcv_h, zeros))

    agg_spec = pl.BlockSpec((_NC, _NODE_BLK, _OD), lambda i: (0, i, 0))
    out = pl.pallas_call(
        _node_body,
        grid=(n_grid,),
        in_specs=[row_spec, agg_spec, agg_spec, _full_spec((_OD, _OD)),
                  _full_spec((_OD, _OD)), _full_spec((1, _OD))],
        out_specs=row_spec,
        out_shape=jax.ShapeDtypeStruct((_N, _OD), f32),
    )(U, aggs[0], aggs[1], eW1b, eW2, eb2_2d)
    return out


# speedup vs baseline: 2.0029x; 1.0407x over previous
import functools

import jax
import jax.numpy as jnp
from jax import lax
from jax.experimental import pallas as pl
from jax.experimental.pallas import tpu as pltpu
from jax.experimental.pallas import tpu_sc as plsc

_N = 10000
_E = 320000
_OD = 128
_RD = 16

_NC = 2
_NS = 16
_NW = _NC * _NS
_NSPLIT = 2
_EH = _E // _NSPLIT
_EW = _EH // _NW
_C = 80
_NCHUNK = _EW // _C
_CT = _EW - _NCHUNK * _C
_ROWS_PER_TILE = 624
_TAIL_ROWS = _N - _NS * _ROWS_PER_TILE


def _pre_body(obj_ref, rw1a_ref, rw1b_ref, rb1_ref, ew1a_ref, eb1_ref,
              p_ref, q_ref, u_ref):
    obj = obj_ref[...]
    p_ref[...] = jnp.dot(obj, rw1a_ref[...], preferred_element_type=jnp.float32)
    q_ref[...] = (jnp.dot(obj, rw1b_ref[...], preferred_element_type=jnp.float32)
                  + rb1_ref[...])
    u_ref[...] = (jnp.dot(obj, ew1a_ref[...], preferred_element_type=jnp.float32)
                  + eb1_ref[...])


def _add_rows(pbuf, qbuf, sbuf, rows):
    def row(r, carry):
        for g in range(_OD // 16):
            sl = pl.ds(g * 16, 16)
            sbuf[r, sl] = pbuf[r, sl] + qbuf[r, sl]
        return carry

    lax.fori_loop(0, rows, row, 0)


def _gather_body(p_hbm, q_hbm, snd_hbm, rcv_hbm, s_hbm,
                 sall, rall, pbuf0, qbuf0, sbuf0, pbuf1, qbuf1, sbuf1,
                 semg0, semg1, semo0, semo1):
    c = lax.axis_index("c")
    s = lax.axis_index("s")
    wid = s * _NC + c
    base = wid * _EW

    pltpu.sync_copy(snd_hbm.at[pl.ds(base, _EW)], sall)
    pltpu.sync_copy(rcv_hbm.at[pl.ds(base, _EW)], rall)

    npair = _NCHUNK // 2

    def pair(i, carry):
        a = 2 * i * _C
        b = a + _C
        cpa_p = pltpu.async_copy(p_hbm.at[sall.at[pl.ds(a, _C)]], pbuf0, semg0)
        cpa_q = pltpu.async_copy(q_hbm.at[rall.at[pl.ds(a, _C)]], qbuf0, semg0)
        cpb_p = pltpu.async_copy(p_hbm.at[sall.at[pl.ds(b, _C)]], pbuf1, semg1)
        cpb_q = pltpu.async_copy(q_hbm.at[rall.at[pl.ds(b, _C)]], qbuf1, semg1)
        cpa_p.wait()
        cpa_q.wait()

        @pl.when(i > 0)
        def _drain0():
            pltpu.make_async_copy(sbuf0, s_hbm.at[pl.ds(base, _C)], semo0).wait()

        _add_rows(pbuf0, qbuf0, sbuf0, _C)
        pltpu.async_copy(sbuf0, s_hbm.at[pl.ds(base + a, _C)], semo0)
        cpb_p.wait()
        cpb_q.wait()

        @pl.when(i > 0)
        def _drain1():
            pltpu.make_async_copy(sbuf1, s_hbm.at[pl.ds(base, _C)], semo1).wait()

        _add_rows(pbuf1, qbuf1, sbuf1, _C)
        pltpu.async_copy(sbuf1, s_hbm.at[pl.ds(base + b, _C)], semo1)
        return carry

    lax.fori_loop(0, npair, pair, 0)
    pltpu.make_async_copy(sbuf0, s_hbm.at[pl.ds(base, _C)], semo0).wait()
    pltpu.make_async_copy(sbuf1, s_hbm.at[pl.ds(base, _C)], semo1).wait()

    toff = _NCHUNK * _C
    cp_p = pltpu.async_copy(p_hbm.at[sall.at[pl.ds(toff, _CT)]],
                            pbuf0.at[pl.ds(0, _CT)], semg0)
    cp_q = pltpu.async_copy(q_hbm.at[rall.at[pl.ds(toff, _CT)]],
                            qbuf0.at[pl.ds(0, _CT)], semg0)
    cp_p.wait()
    cp_q.wait()
    _add_rows(pbuf0, qbuf0, sbuf0, _CT)
    pltpu.sync_copy(sbuf0.at[pl.ds(0, _CT)], s_hbm.at[pl.ds(base + toff, _CT)])


def _edge_body(s_ref, rel_ref, w1c_ref, w2_ref, b2_ref, out_ref):
    x = (s_ref[...]
         + jnp.dot(rel_ref[...], w1c_ref[...], preferred_element_type=jnp.float32))
    h = jnp.maximum(x, 0.0)
    out_ref[...] = (jnp.dot(h, w2_ref[...], preferred_element_type=jnp.float32)
                    + b2_ref[...])


def _scatter_body(eff_hbm, rcv_hbm, zeros_hbm, out_hbm,
                  ridx0, ridx1, ridx2, ridx3, ridx_t,
                  ebuf0, ebuf1, ebuf2, ebuf3, acc,
                  seml0, seml1, seml2, seml3,
                  sema0, sema1, sema2, sema3):
    c = lax.axis_index("c")
    s = lax.axis_index("s")
    wid = s * _NC + c

    pltpu.sync_copy(zeros_hbm.at[pl.ds(s * _ROWS_PER_TILE, _ROWS_PER_TILE)],
                    acc.at[pl.ds(s * _ROWS_PER_TILE, _ROWS_PER_TILE)])

    @pl.when(s == _NS - 1)
    def _zero_tail():
        pltpu.sync_copy(zeros_hbm.at[pl.ds(_NS * _ROWS_PER_TILE, _TAIL_ROWS)],
                        acc.at[pl.ds(_NS * _ROWS_PER_TILE, _TAIL_ROWS)])

    plsc.subcore_barrier()

    base = wid * _EW
    nquad = _NCHUNK // 4

    def quad(i, carry):
        o0 = base + 4 * i * _C
        o1 = o0 + _C
        o2 = o1 + _C
        o3 = o2 + _C

        @pl.when(i > 0)
        def _drain01():
            pltpu.make_async_copy(ebuf0, acc.at[ridx0], sema0).wait()
            pltpu.make_async_copy(ebuf1, acc.at[ridx1], sema1).wait()

        pltpu.sync_copy(rcv_hbm.at[pl.ds(o0, _C)], ridx0)
        cp0 = pltpu.async_copy(eff_hbm.at[pl.ds(o0, _C)], ebuf0, seml0)
        pltpu.sync_copy(rcv_hbm.at[pl.ds(o1, _C)], ridx1)
        cp1 = pltpu.async_copy(eff_hbm.at[pl.ds(o1, _C)], ebuf1, seml1)

        @pl.when(i > 0)
        def _drain23():
            pltpu.make_async_copy(ebuf2, acc.at[ridx2], sema2).wait()
            pltpu.make_async_copy(ebuf3, acc.at[ridx3], sema3).wait()

        pltpu.sync_copy(rcv_hbm.at[pl.ds(o2, _C)], ridx2)
        cp2 = pltpu.async_copy(eff_hbm.at[pl.ds(o2, _C)], ebuf2, seml2)
        pltpu.sync_copy(rcv_hbm.at[pl.ds(o3, _C)], ridx3)
        cp3 = pltpu.async_copy(eff_hbm.at[pl.ds(o3, _C)], ebuf3, seml3)

        cp0.wait()
        pltpu.async_copy(ebuf0, acc.at[ridx0], sema0, add=True)
        cp1.wait()
        pltpu.async_copy(ebuf1, acc.at[ridx1], sema1, add=True)
        cp2.wait()
        pltpu.async_copy(ebuf2, acc.at[ridx2], sema2, add=True)
        cp3.wait()
        pltpu.async_copy(ebuf3, acc.at[ridx3], sema3, add=True)
        return carry

    lax.fori_loop(0, nquad, quad, 0)
    pltpu.make_async_copy(ebuf0, acc.at[ridx0], sema0).wait()
    pltpu.make_async_copy(ebuf1, acc.at[ridx1], sema1).wait()
    pltpu.make_async_copy(ebuf2, acc.at[ridx2], sema2).wait()
    pltpu.make_async_copy(ebuf3, acc.at[ridx3], sema3).wait()

    for k in range(4 * nquad, _NCHUNK):
        off = base + k * _C
        pltpu.sync_copy(rcv_hbm.at[pl.ds(off, _C)], ridx0)
        pltpu.sync_copy(eff_hbm.at[pl.ds(off, _C)], ebuf0)
        pltpu.sync_copy(ebuf0, acc.at[ridx0], add=True)

    toff = base + _NCHUNK * _C
    pltpu.sync_copy(rcv_hbm.at[pl.ds(toff, _CT)], ridx_t)
    pltpu.sync_copy(eff_hbm.at[pl.ds(toff, _CT)], ebuf0.at[pl.ds(0, _CT)])
    pltpu.sync_copy(ebuf0.at[pl.ds(0, _CT)], acc.at[ridx_t], add=True)
    plsc.subcore_barrier()

    pltpu.sync_copy(acc.at[pl.ds(s * _ROWS_PER_TILE, _ROWS_PER_TILE)],
                    out_hbm.at[c, pl.ds(s * _ROWS_PER_TILE, _ROWS_PER_TILE)])

    @pl.when(s == _NS - 1)
    def _flush_tail():
        pltpu.sync_copy(acc.at[pl.ds(_NS * _ROWS_PER_TILE, _TAIL_ROWS)],
                        out_hbm.at[c, pl.ds(_NS * _ROWS_PER_TILE, _TAIL_ROWS)])


def _node_body(u_ref, a_ref, b_ref, ew1b_ref, ew2_ref, eb2_ref, out_ref):
    agg = a_ref[0] + a_ref[1] + b_ref[0] + b_ref[1]
    x = u_ref[...] + jnp.dot(agg, ew1b_ref[...], preferred_element_type=jnp.float32)
    h = jnp.maximum(x, 0.0)
    out_ref[...] = (jnp.dot(h, ew2_ref[...], preferred_element_type=jnp.float32)
                    + eb2_ref[...])


_NODE_BLK = 2000
_EDGE_BLK = 2000


def _full_spec(shape):
    return pl.BlockSpec(shape, lambda i: tuple(0 for _ in shape))


def kernel(objects, relations, senders, receivers,
           rW1, rb1, rW2, rb2,
           oW1, ob1, oW2, ob2,
           eW1, eb1, eW2, eb2):
    f32 = jnp.float32
    rW1a = rW1[:_OD]
    rW1b = rW1[_OD:2 * _OD]
    rW1c = rW1[2 * _OD:]
    eW1a = eW1[:_OD]
    eW1b = eW1[_OD:]
    rb1_2d = rb1.reshape(1, _OD)
    rb2_2d = rb2.reshape(1, _OD)
    eb1_2d = eb1.reshape(1, _OD)
    eb2_2d = eb2.reshape(1, _OD)

    n_grid = _N // _NODE_BLK
    row_spec = pl.BlockSpec((_NODE_BLK, _OD), lambda i: (i, 0))
    P, Q, U = pl.pallas_call(
        _pre_body,
        grid=(n_grid,),
        in_specs=[row_spec, _full_spec((_OD, _OD)), _full_spec((_OD, _OD)),
                  _full_spec((1, _OD)), _full_spec((_OD, _OD)),
                  _full_spec((1, _OD))],
        out_specs=[row_spec, row_spec, row_spec],
        out_shape=[jax.ShapeDtypeStruct((_N, _OD), f32)] * 3,
    )(objects, rW1a, rW1b, rb1_2d, eW1a, eb1_2d)

    mesh = plsc.VectorSubcoreMesh(core_axis_name="c", subcore_axis_name="s")
    gather = functools.partial(
        pl.kernel,
        mesh=mesh,
        out_type=jax.ShapeDtypeStruct((_EH, _OD), f32),
        scratch_types=[
            pltpu.VMEM((_EW,), jnp.int32),
            pltpu.VMEM((_EW,), jnp.int32),
            pltpu.VMEM((_C, _OD), f32),
            pltpu.VMEM((_C, _OD), f32),
            pltpu.VMEM((_C, _OD), f32),
            pltpu.VMEM((_C, _OD), f32),
            pltpu.VMEM((_C, _OD), f32),
            pltpu.VMEM((_C, _OD), f32),
            pltpu.SemaphoreType.DMA,
            pltpu.SemaphoreType.DMA,
            pltpu.SemaphoreType.DMA,
            pltpu.SemaphoreType.DMA,
        ],
    )(_gather_body)

    scatter = functools.partial(
        pl.kernel,
        mesh=mesh,
        out_type=jax.ShapeDtypeStruct((_NC, _N, _OD), f32),
        scratch_types=(
            [pltpu.VMEM((_C,), jnp.int32)] * 4
            + [pltpu.VMEM((_CT,), jnp.int32)]
            + [pltpu.VMEM((_C, _OD), f32)] * 4
            + [pltpu.VMEM_SHARED((_N, _OD), f32)]
            + [pltpu.SemaphoreType.DMA] * 8
        ),
    )(_scatter_body)

    e_grid = _EH // _EDGE_BLK
    erow_spec = pl.BlockSpec((_EDGE_BLK, _OD), lambda i: (i, 0))
    rel_spec = pl.BlockSpec((_EDGE_BLK, _RD), lambda i: (i, 0))
    edge_mlp = pl.pallas_call(
        _edge_body,
        grid=(e_grid,),
        in_specs=[erow_spec, rel_spec, _full_spec((_RD, _OD)),
                  _full_spec((_OD, _OD)), _full_spec((1, _OD))],
        out_specs=erow_spec,
        out_shape=jax.ShapeDtypeStruct((_EH, _OD), f32),
    )

    zeros = jnp.zeros((_N, _OD), f32)
    aggs = []
    for h in range(_NSPLIT):
        lo, hi = h * _EH, (h + 1) * _EH
        snd_h = senders[lo:hi]
        rcv_h = receivers[lo:hi]
        S = gather(P, Q, snd_h, rcv_h)
        eff = edge_mlp(S, relations[lo:hi], rW1c, rW2, rb2_2d)
        aggs.append(scatter(eff, rcv_h, zeros))

    agg_spec = pl.BlockSpec((_NC, _NODE_BLK, _OD), lambda i: (0, i, 0))
    out = pl.pallas_call(
        _node_body,
        grid=(n_grid,),
        in_specs=[row_spec, agg_spec, agg_spec, _full_spec((_OD, _OD)),
                  _full_spec((_OD, _OD)), _full_spec((1, _OD))],
        out_specs=row_spec,
        out_shape=jax.ShapeDtypeStruct((_N, _OD), f32),
    )(U, aggs[0], aggs[1], eW1b, eW2, eb2_2d)
    return out


# speedup vs baseline: 2.0066x; 1.0018x over previous
import functools

import jax
import jax.numpy as jnp
from jax import lax
from jax.experimental import pallas as pl
from jax.experimental.pallas import tpu as pltpu
from jax.experimental.pallas import tpu_sc as plsc

_N = 10000
_E = 320000
_OD = 128
_RD = 16

_NC = 2
_NS = 16
_NW = _NC * _NS
_NSPLIT = 2
_EH = _E // _NSPLIT
_EW = _EH // _NW
_C = 80
_NCHUNK = _EW // _C
_CT = _EW - _NCHUNK * _C
_CG = 128
_NCG = _EW // _CG
_CGT = _EW - _NCG * _CG
_ROWS_PER_TILE = 624
_TAIL_ROWS = _N - _NS * _ROWS_PER_TILE


def _pre_body(obj_ref, rw1a_ref, rw1b_ref, rb1_ref, ew1a_ref, eb1_ref,
              p_ref, q_ref, u_ref):
    obj = obj_ref[...]
    p_ref[...] = jnp.dot(obj, rw1a_ref[...], preferred_element_type=jnp.float32)
    q_ref[...] = (jnp.dot(obj, rw1b_ref[...], preferred_element_type=jnp.float32)
                  + rb1_ref[...])
    u_ref[...] = (jnp.dot(obj, ew1a_ref[...], preferred_element_type=jnp.float32)
                  + eb1_ref[...])


def _add_rows(pbuf, qbuf, sbuf, rows):
    def row(r, carry):
        for g in range(_OD // 16):
            sl = pl.ds(g * 16, 16)
            sbuf[r, sl] = pbuf[r, sl] + qbuf[r, sl]
        return carry

    lax.fori_loop(0, rows, row, 0)


def _gather_body(p_hbm, q_hbm, snd_hbm, rcv_hbm, s_hbm,
                 sall, rall, pbuf0, qbuf0, sbuf0, pbuf1, qbuf1, sbuf1,
                 semg0, semg1, semo0, semo1):
    c = lax.axis_index("c")
    s = lax.axis_index("s")
    wid = s * _NC + c
    base = wid * _EW

    pltpu.sync_copy(snd_hbm.at[pl.ds(base, _EW)], sall)
    pltpu.sync_copy(rcv_hbm.at[pl.ds(base, _EW)], rall)

    npair = _NCG // 2

    def pair(i, carry):
        a = 2 * i * _CG
        b = a + _CG
        cpa_p = pltpu.async_copy(p_hbm.at[sall.at[pl.ds(a, _CG)]], pbuf0, semg0)
        cpa_q = pltpu.async_copy(q_hbm.at[rall.at[pl.ds(a, _CG)]], qbuf0, semg0)
        cpb_p = pltpu.async_copy(p_hbm.at[sall.at[pl.ds(b, _CG)]], pbuf1, semg1)
        cpb_q = pltpu.async_copy(q_hbm.at[rall.at[pl.ds(b, _CG)]], qbuf1, semg1)
        cpa_p.wait()
        cpa_q.wait()

        @pl.when(i > 0)
        def _drain0():
            pltpu.make_async_copy(sbuf0, s_hbm.at[pl.ds(base, _CG)], semo0).wait()

        _add_rows(pbuf0, qbuf0, sbuf0, _CG)
        pltpu.async_copy(sbuf0, s_hbm.at[pl.ds(base + a, _CG)], semo0)
        cpb_p.wait()
        cpb_q.wait()

        @pl.when(i > 0)
        def _drain1():
            pltpu.make_async_copy(sbuf1, s_hbm.at[pl.ds(base, _CG)], semo1).wait()

        _add_rows(pbuf1, qbuf1, sbuf1, _CG)
        pltpu.async_copy(sbuf1, s_hbm.at[pl.ds(base + b, _CG)], semo1)
        return carry

    lax.fori_loop(0, npair, pair, 0)
    pltpu.make_async_copy(sbuf0, s_hbm.at[pl.ds(base, _CG)], semo0).wait()
    pltpu.make_async_copy(sbuf1, s_hbm.at[pl.ds(base, _CG)], semo1).wait()

    loff = 2 * npair * _CG
    cp_p = pltpu.async_copy(p_hbm.at[sall.at[pl.ds(loff, _CG)]], pbuf0, semg0)
    cp_q = pltpu.async_copy(q_hbm.at[rall.at[pl.ds(loff, _CG)]], qbuf0, semg0)
    toff = loff + _CG
    cp_tp = pltpu.async_copy(p_hbm.at[sall.at[pl.ds(toff, _CGT)]],
                             pbuf1.at[pl.ds(0, _CGT)], semg1)
    cp_tq = pltpu.async_copy(q_hbm.at[rall.at[pl.ds(toff, _CGT)]],
                             qbuf1.at[pl.ds(0, _CGT)], semg1)
    cp_p.wait()
    cp_q.wait()
    _add_rows(pbuf0, qbuf0, sbuf0, _CG)
    pltpu.sync_copy(sbuf0, s_hbm.at[pl.ds(base + loff, _CG)])
    cp_tp.wait()
    cp_tq.wait()
    _add_rows(pbuf1, qbuf1, sbuf1, _CGT)
    pltpu.sync_copy(sbuf1.at[pl.ds(0, _CGT)], s_hbm.at[pl.ds(base + toff, _CGT)])


def _edge_body(s_ref, rel_ref, w1c_ref, w2_ref, b2_ref, out_ref):
    x = (s_ref[...]
         + jnp.dot(rel_ref[...], w1c_ref[...], preferred_element_type=jnp.float32))
    h = jnp.maximum(x, 0.0)
    out_ref[...] = (jnp.dot(h, w2_ref[...], preferred_element_type=jnp.float32)
                    + b2_ref[...])


def _scatter_body(eff_hbm, rcv_hbm, zeros_hbm, out_hbm,
                  ridx0, ridx1, ridx2, ridx3, ridx_t,
                  ebuf0, ebuf1, ebuf2, ebuf3, acc,
                  seml0, seml1, seml2, seml3,
                  sema0, sema1, sema2, sema3):
    c = lax.axis_index("c")
    s = lax.axis_index("s")
    wid = s * _NC + c

    pltpu.sync_copy(zeros_hbm.at[pl.ds(s * _ROWS_PER_TILE, _ROWS_PER_TILE)],
                    acc.at[pl.ds(s * _ROWS_PER_TILE, _ROWS_PER_TILE)])

    @pl.when(s == _NS - 1)
    def _zero_tail():
        pltpu.sync_copy(zeros_hbm.at[pl.ds(_NS * _ROWS_PER_TILE, _TAIL_ROWS)],
                        acc.at[pl.ds(_NS * _ROWS_PER_TILE, _TAIL_ROWS)])

    plsc.subcore_barrier()

    base = wid * _EW
    nquad = _NCHUNK // 4

    def quad(i, carry):
        o0 = base + 4 * i * _C
        o1 = o0 + _C
        o2 = o1 + _C
        o3 = o2 + _C

        @pl.when(i > 0)
        def _drain01():
            pltpu.make_async_copy(ebuf0, acc.at[ridx0], sema0).wait()
            pltpu.make_async_copy(ebuf1, acc.at[ridx1], sema1).wait()

        pltpu.sync_copy(rcv_hbm.at[pl.ds(o0, _C)], ridx0)
        cp0 = pltpu.async_copy(eff_hbm.at[pl.ds(o0, _C)], ebuf0, seml0)
        pltpu.sync_copy(rcv_hbm.at[pl.ds(o1, _C)], ridx1)
        cp1 = pltpu.async_copy(eff_hbm.at[pl.ds(o1, _C)], ebuf1, seml1)

        @pl.when(i > 0)
        def _drain23():
            pltpu.make_async_copy(ebuf2, acc.at[ridx2], sema2).wait()
            pltpu.make_async_copy(ebuf3, acc.at[ridx3], sema3).wait()

        pltpu.sync_copy(rcv_hbm.at[pl.ds(o2, _C)], ridx2)
        cp2 = pltpu.async_copy(eff_hbm.at[pl.ds(o2, _C)], ebuf2, seml2)
        pltpu.sync_copy(rcv_hbm.at[pl.ds(o3, _C)], ridx3)
        cp3 = pltpu.async_copy(eff_hbm.at[pl.ds(o3, _C)], ebuf3, seml3)

        cp0.wait()
        pltpu.async_copy(ebuf0, acc.at[ridx0], sema0, add=True)
        cp1.wait()
        pltpu.async_copy(ebuf1, acc.at[ridx1], sema1, add=True)
        cp2.wait()
        pltpu.async_copy(ebuf2, acc.at[ridx2], sema2, add=True)
        cp3.wait()
        pltpu.async_copy(ebuf3, acc.at[ridx3], sema3, add=True)
        return carry

    lax.fori_loop(0, nquad, quad, 0)
    pltpu.make_async_copy(ebuf0, acc.at[ridx0], sema0).wait()
    pltpu.make_async_copy(ebuf1, acc.at[ridx1], sema1).wait()
    pltpu.make_async_copy(ebuf2, acc.at[ridx2], sema2).wait()
    pltpu.make_async_copy(ebuf3, acc.at[ridx3], sema3).wait()

    for k in range(4 * nquad, _NCHUNK):
        off = base + k * _C
        pltpu.sync_copy(rcv_hbm.at[pl.ds(off, _C)], ridx0)
        pltpu.sync_copy(eff_hbm.at[pl.ds(off, _C)], ebuf0)
        pltpu.sync_copy(ebuf0, acc.at[ridx0], add=True)

    toff = base + _NCHUNK * _C
    pltpu.sync_copy(rcv_hbm.at[pl.ds(toff, _CT)], ridx_t)
    pltpu.sync_copy(eff_hbm.at[pl.ds(toff, _CT)], ebuf0.at[pl.ds(0, _CT)])
    pltpu.sync_copy(ebuf0.at[pl.ds(0, _CT)], acc.at[ridx_t], add=True)
    plsc.subcore_barrier()

    pltpu.sync_copy(acc.at[pl.ds(s * _ROWS_PER_TILE, _ROWS_PER_TILE)],
                    out_hbm.at[c, pl.ds(s * _ROWS_PER_TILE, _ROWS_PER_TILE)])

    @pl.when(s == _NS - 1)
    def _flush_tail():
        pltpu.sync_copy(acc.at[pl.ds(_NS * _ROWS_PER_TILE, _TAIL_ROWS)],
                        out_hbm.at[c, pl.ds(_NS * _ROWS_PER_TILE, _TAIL_ROWS)])


def _node_body(u_ref, a_ref, b_ref, ew1b_ref, ew2_ref, eb2_ref, out_ref):
    agg = a_ref[0] + a_ref[1] + b_ref[0] + b_ref[1]
    x = u_ref[...] + jnp.dot(agg, ew1b_ref[...], preferred_element_type=jnp.float32)
    h = jnp.maximum(x, 0.0)
    out_ref[...] = (jnp.dot(h, ew2_ref[...], preferred_element_type=jnp.float32)
                    + eb2_ref[...])


_NODE_BLK = 2000
_EDGE_BLK = 2000


def _full_spec(shape):
    return pl.BlockSpec(shape, lambda i: tuple(0 for _ in shape))


def kernel(objects, relations, senders, receivers,
           rW1, rb1, rW2, rb2,
           oW1, ob1, oW2, ob2,
           eW1, eb1, eW2, eb2):
    f32 = jnp.float32
    rW1a = rW1[:_OD]
    rW1b = rW1[_OD:2 * _OD]
    rW1c = rW1[2 * _OD:]
    eW1a = eW1[:_OD]
    eW1b = eW1[_OD:]
    rb1_2d = rb1.reshape(1, _OD)
    rb2_2d = rb2.reshape(1, _OD)
    eb1_2d = eb1.reshape(1, _OD)
    eb2_2d = eb2.reshape(1, _OD)

    n_grid = _N // _NODE_BLK
    row_spec = pl.BlockSpec((_NODE_BLK, _OD), lambda i: (i, 0))
    P, Q, U = pl.pallas_call(
        _pre_body,
        grid=(n_grid,),
        in_specs=[row_spec, _full_spec((_OD, _OD)), _full_spec((_OD, _OD)),
                  _full_spec((1, _OD)), _full_spec((_OD, _OD)),
                  _full_spec((1, _OD))],
        out_specs=[row_spec, row_spec, row_spec],
        out_shape=[jax.ShapeDtypeStruct((_N, _OD), f32)] * 3,
    )(objects, rW1a, rW1b, rb1_2d, eW1a, eb1_2d)

    mesh = plsc.VectorSubcoreMesh(core_axis_name="c", subcore_axis_name="s")
    gather = functools.partial(
        pl.kernel,
        mesh=mesh,
        out_type=jax.ShapeDtypeStruct((_EH, _OD), f32),
        scratch_types=(
            [pltpu.VMEM((_EW,), jnp.int32)] * 2
            + [pltpu.VMEM((_CG, _OD), f32)] * 6
            + [pltpu.SemaphoreType.DMA] * 4
        ),
    )(_gather_body)

    scatter = functools.partial(
        pl.kernel,
        mesh=mesh,
        out_type=jax.ShapeDtypeStruct((_NC, _N, _OD), f32),
        scratch_types=(
            [pltpu.VMEM((_C,), jnp.int32)] * 4
            + [pltpu.VMEM((_CT,), jnp.int32)]
            + [pltpu.VMEM((_C, _OD), f32)] * 4
            + [pltpu.VMEM_SHARED((_N, _OD), f32)]
            + [pltpu.SemaphoreType.DMA] * 8
        ),
    )(_scatter_body)

    e_grid = _EH // _EDGE_BLK
    erow_spec = pl.BlockSpec((_EDGE_BLK, _OD), lambda i: (i, 0))
    rel_spec = pl.BlockSpec((_EDGE_BLK, _RD), lambda i: (i, 0))
    edge_mlp = pl.pallas_call(
        _edge_body,
        grid=(e_grid,),
        in_specs=[erow_spec, rel_spec, _full_spec((_RD, _OD)),
                  _full_spec((_OD, _OD)), _full_spec((1, _OD))],
        out_specs=erow_spec,
        out_shape=jax.ShapeDtypeStruct((_EH, _OD), f32),
    )

    zeros = jnp.zeros((_N, _OD), f32)
    aggs = []
    for h in range(_NSPLIT):
        lo, hi = h * _EH, (h + 1) * _EH
        snd_h = senders[lo:hi]
        rcv_h = receivers[lo:hi]
        S = gather(P, Q, snd_h, rcv_h)
        eff = edge_mlp(S, relations[lo:hi], rW1c, rW2, rb2_2d)
        aggs.append(scatter(eff, rcv_h, zeros))

    agg_spec = pl.BlockSpec((_NC, _NODE_BLK, _OD), lambda i: (0, i, 0))
    out = pl.pallas_call(
        _node_body,
        grid=(n_grid,),
        in_specs=[row_spec, agg_spec, agg_spec, _full_spec((_OD, _OD)),
                  _full_spec((_OD, _OD)), _full_spec((1, _OD))],
        out_specs=row_spec,
        out_shape=jax.ShapeDtypeStruct((_N, _OD), f32),
    )(U, aggs[0], aggs[1], eW1b, eW2, eb2_2d)
    return out
